# baseline (speedup 1.0000x reference)
import jax
import jax.numpy as jnp
from jax import lax
from jax.experimental import pallas as pl
from jax.experimental.pallas import tpu as pltpu
from jax.experimental.pallas import tpu_sc as plsc

_K = 32
_BITS_TWO = 0x40000000
_N = 32768
_NSUB = 16
_CHUNK = _N // _NSUB
_VPC = _CHUNK // 16


def _iota16():
    return lax.iota(jnp.int32, 16)


def _splat(s):
    return jnp.full((16,), s, jnp.int32)


def _sc_body(x_hbm, y_hbm, xv, kv, yv, hist_v, hall_v, tcv, trd,
             hshared, tshared):
    wid = lax.axis_index("s")
    base = wid * _CHUNK
    zeros16 = jnp.zeros((16,), jnp.int32)
    ones16 = jnp.ones((16,), jnp.int32)

    pltpu.sync_copy(x_hbm.at[pl.ds(base, _CHUNK)], xv)

    def keys_body(i, _):
        xs = xv[pl.ds(i * 16, 16)]
        kv[pl.ds(i * 16, 16)] = lax.bitcast_convert_type(
            jnp.maximum(xs, 2.0), jnp.int32)
        return 0

    lax.fori_loop(0, _VPC, keys_body, 0)

    pref = jnp.int32(0)
    c_above = jnp.int32(0)

    for r in range(4):
        shift = 24 - 8 * r
        for i in range(16):
            hist_v[pl.ds(i * 16, 16)] = zeros16

        pref_s = _splat(pref)

        def hist_body(i, _, shift=shift, pref_s=pref_s, first=(r == 0)):
            ks = kv[pl.ds(i * 16, 16)]
            h = lax.shift_right_arithmetic(ks, _splat(shift))
            b = jnp.bitwise_and(h, _splat(255))
            if first:
                plsc.addupdate_scatter(hist_v, [b], ones16)
            else:
                valid = lax.shift_right_arithmetic(h, _splat(8)) == pref_s
                plsc.addupdate_scatter(hist_v, [b], ones16, mask=valid)
            return 0

        lax.fori_loop(0, _VPC, hist_body, 0)

        pltpu.sync_copy(hist_v, hshared.at[wid])
        plsc.subcore_barrier()
        pltpu.sync_copy(hshared, hall_v)

        found = jnp.bool_(False)
        sel = jnp.int32(0)
        for j in range(15, -1, -1):
            rowv = zeros16
            for rr in range(_NSUB):
                rowv = rowv + hall_v[rr, pl.ds(j * 16, 16)]
            rev = lax.rev(rowv, (0,))
            cs = plsc.cumsum(rev)
            cond = (_splat(c_above) + cs) >= _K
            npos = jnp.max(plsc.all_reduce_population_count(cond))
            found_this = jnp.logical_and(jnp.logical_not(found), npos > 0)
            lane = 16 - npos
            lane_m = _iota16() == lane
            cs_at = jnp.sum(jnp.where(lane_m, cs, 0))
            rv_at = jnp.sum(jnp.where(lane_m, rev, 0))
            row_total = jnp.sum(rowv)
            sel = jnp.where(found_this, j * 16 + (15 - lane), sel)
            c_above = jnp.where(
                found, c_above,
                jnp.where(found_this, c_above + cs_at - rv_at,
                          c_above + row_total))
            found = jnp.logical_or(found, npos > 0)
        pref = pref * 256 + sel

    t = pref
    c_gt = c_above
    m_eff = jnp.where(t == _BITS_TWO, 0, _K - c_gt)
    t_s = _splat(t)

    def tiecnt_body(i, c):
        ts = (kv[pl.ds(i * 16, 16)] == t_s).astype(jnp.int32)
        return c + jnp.sum(ts)

    my_ties = lax.fori_loop(0, _VPC, tiecnt_body, jnp.int32(0))
    tcv[:] = _splat(my_ties)
    pltpu.sync_copy(tcv, tshared.at[wid])
    plsc.subcore_barrier()
    pltpu.sync_copy(tshared, trd)
    offv = zeros16
    for rrow in range(_NSUB):
        offv = offv + jnp.where(wid > rrow, trd[rrow, :], zeros16)
    tie_off = jnp.max(offv)

    m_s = _splat(m_eff)

    def out_body(i, run):
        ks = kv[pl.ds(i * 16, 16)]
        gt = ks > t_s
        tie = ks == t_s
        tin = tie.astype(jnp.int32)
        excl = plsc.cumsum(tin) - tin
        grank = _splat(tie_off + run) + excl
        sel_t = jnp.logical_and(tie, grank < m_s)
        yv[pl.ds(i * 16, 16)] = jnp.where(
            jnp.logical_or(gt, sel_t), 1.0, 0.0).astype(jnp.float32)
        return run + jnp.sum(tin)

    lax.fori_loop(0, _VPC, out_body, jnp.int32(0))
    pltpu.sync_copy(yv, y_hbm.at[pl.ds(base, _CHUNK)])


def kernel(x):
    mesh = plsc.VectorSubcoreMesh(
        core_axis_name="c", subcore_axis_name="s", num_cores=1,
        num_subcores=_NSUB)
    y = pl.kernel(
        _sc_body,
        out_type=jax.ShapeDtypeStruct((_N,), jnp.float32),
        mesh=mesh,
        compiler_params=pltpu.CompilerParams(needs_layout_passes=False),
        scratch_types=[
            pltpu.VMEM((_CHUNK,), jnp.float32),
            pltpu.VMEM((_CHUNK,), jnp.int32),
            pltpu.VMEM((_CHUNK,), jnp.float32),
            pltpu.VMEM((256,), jnp.int32),
            pltpu.VMEM((_NSUB, 256), jnp.int32),
            pltpu.VMEM((16,), jnp.int32),
            pltpu.VMEM((_NSUB, 16), jnp.int32),
            pltpu.VMEM_SHARED((_NSUB, 256), jnp.int32),
            pltpu.VMEM_SHARED((_NSUB, 16), jnp.int32),
        ],
    )(x)
    return y


# speedup vs baseline: 1.0011x; 1.0011x over previous
import jax
import jax.numpy as jnp
from jax import lax
from jax.experimental import pallas as pl
from jax.experimental.pallas import tpu as pltpu
from jax.experimental.pallas import tpu_sc as plsc

_K = 32
_BITS_TWO = 0x40000000
_N = 32768
_NSUB = 16
_CHUNK = _N // _NSUB
_VPC = _CHUNK // 16


def _iota16():
    return lax.iota(jnp.int32, 16)


def _splat(s):
    return jnp.full((16,), s, jnp.int32)


def _sc_body(x_hbm, y_hbm, xv, kv, yv, hist_v, hall_v, tcv, trd,
             hshared, tshared):
    wid = lax.axis_index("s")
    base = wid * _CHUNK
    zeros16 = jnp.zeros((16,), jnp.int32)
    ones16 = jnp.ones((16,), jnp.int32)

    pltpu.sync_copy(x_hbm.at[pl.ds(base, _CHUNK)], xv)

    def keys_body(i, _):
        xs = xv[pl.ds(i * 16, 16)]
        kv[pl.ds(i * 16, 16)] = lax.bitcast_convert_type(
            jnp.maximum(xs, 2.0), jnp.int32)
        return 0

    lax.fori_loop(0, _VPC, keys_body, 0)

    pref = jnp.int32(0)
    c_above = jnp.int32(0)

    for r in range(4):
        shift = 24 - 8 * r
        for i in range(16):
            hist_v[pl.ds(i * 16, 16)] = zeros16

        pref_s = _splat(pref)

        def hist_body(i, _, shift=shift, pref_s=pref_s, first=(r == 0)):
            ks = kv[pl.ds(i * 16, 16)]
            h = lax.shift_right_arithmetic(ks, _splat(shift))
            b = jnp.bitwise_and(h, _splat(255))
            if first:
                plsc.addupdate_scatter(hist_v, [b], ones16)
            else:
                valid = lax.shift_right_arithmetic(h, _splat(8)) == pref_s
                plsc.addupdate_scatter(hist_v, [b], ones16, mask=valid)
            return 0

        lax.fori_loop(0, _VPC, hist_body, 0)

        pltpu.sync_copy(hist_v, hshared.at[wid])
        plsc.subcore_barrier()
        pltpu.sync_copy(hshared, hall_v)

        found = jnp.bool_(False)
        sel = jnp.int32(0)
        for j in range(15, -1, -1):
            rowv = zeros16
            for rr in range(_NSUB):
                rowv = rowv + hall_v[rr, pl.ds(j * 16, 16)]
            rev = lax.rev(rowv, (0,))
            cs = plsc.cumsum(rev)
            cond = (_splat(c_above) + cs) >= _K
            npos = jnp.max(plsc.all_reduce_population_count(cond))
            found_this = jnp.logical_and(jnp.logical_not(found), npos > 0)
            lane = 16 - npos
            lane_m = _iota16() == lane
            cs_at = jnp.sum(jnp.where(lane_m, cs, 0))
            rv_at = jnp.sum(jnp.where(lane_m, rev, 0))
            row_total = jnp.sum(rowv)
            sel = jnp.where(found_this, j * 16 + (15 - lane), sel)
            c_above = jnp.where(
                found, c_above,
                jnp.where(found_this, c_above + cs_at - rv_at,
                          c_above + row_total))
            found = jnp.logical_or(found, npos > 0)
        pref = pref * 256 + sel

    t = pref
    c_gt = c_above
    m_eff = jnp.where(t == _BITS_TWO, 0, _K - c_gt)
    t_s = _splat(t)

    def tiecnt_body(i, c):
        ts = (kv[pl.ds(i * 16, 16)] == t_s).astype(jnp.int32)
        return c + jnp.sum(ts)

    my_ties = lax.fori_loop(0, _VPC, tiecnt_body, jnp.int32(0))
    tcv[:] = _splat(my_ties)
    pltpu.sync_copy(tcv, tshared.at[wid])
    plsc.subcore_barrier()
    pltpu.sync_copy(tshared, trd)
    offv = zeros16
    for rrow in range(_NSUB):
        offv = offv + jnp.where(wid > rrow, trd[rrow, :], zeros16)
    tie_off = jnp.max(offv)

    m_s = _splat(m_eff)

    def out_body(i, run):
        ks = kv[pl.ds(i * 16, 16)]
        gt = ks > t_s
        tie = ks == t_s
        tin = tie.astype(jnp.int32)
        excl = plsc.cumsum(tin) - tin
        grank = _splat(tie_off + run) + excl
        sel_t = jnp.logical_and(tie, grank < m_s)
        yv[pl.ds(i * 16, 16)] = jnp.where(
            jnp.logical_or(gt, sel_t), 1.0, 0.0).astype(jnp.float32)
        return run + jnp.sum(tin)

    lax.fori_loop(0, _VPC, out_body, jnp.int32(0))
    pltpu.sync_copy(yv, y_hbm.at[pl.ds(base, _CHUNK)])


def kernel(x):
    mesh = plsc.VectorSubcoreMesh(
        core_axis_name="c", subcore_axis_name="s", num_cores=1,
        num_subcores=_NSUB)
    y = pl.kernel(
        _sc_body,
        out_type=jax.ShapeDtypeStruct((_N,), jnp.float32),
        mesh=mesh,
        compiler_params=pltpu.CompilerParams(
            needs_layout_passes=False, skip_device_barrier=True),
        scratch_types=[
            pltpu.VMEM((_CHUNK,), jnp.float32),
            pltpu.VMEM((_CHUNK,), jnp.int32),
            pltpu.VMEM((_CHUNK,), jnp.float32),
            pltpu.VMEM((256,), jnp.int32),
            pltpu.VMEM((_NSUB, 256), jnp.int32),
            pltpu.VMEM((16,), jnp.int32),
            pltpu.VMEM((_NSUB, 16), jnp.int32),
            pltpu.VMEM_SHARED((_NSUB, 256), jnp.int32),
            pltpu.VMEM_SHARED((_NSUB, 16), jnp.int32),
        ],
    )(x)
    return y


# speedup vs baseline: 3.7545x; 3.7504x over previous
import jax
import jax.numpy as jnp
from jax import lax
from jax.experimental import pallas as pl

_K = 32
_BITS_TWO = 0x40000000
_BITS_INF = 0x7F800000
_N = 32768
_ROWS, _COLS = 256, 128


def _body(x_ref, y_ref):
    x = x_ref[...]
    ones_row = jnp.ones((1, _ROWS), jnp.float32)

    def count_ge(mask_f):
        part = lax.dot_general(ones_row, mask_f, (((1,), (0,)), ((), ())),
                               preferred_element_type=jnp.float32)
        return jnp.sum(part)

    def val_step(_, lohi):
        lo, hi = lohi
        mid = lo + (hi - lo) // 2
        mid_f = lax.bitcast_convert_type(mid, jnp.float32)
        c = count_ge(jnp.where(x >= mid_f, 1.0, 0.0))
        big = c >= _K
        return jnp.where(big, mid, lo), jnp.where(big, hi, mid)

    lo, _ = lax.fori_loop(0, 30, val_step,
                          (jnp.int32(_BITS_TWO), jnp.int32(_BITS_INF)))
    t = lo
    t_f = lax.bitcast_convert_type(t, jnp.float32)

    gt = x > t_f
    c_gt = count_ge(jnp.where(gt, 1.0, 0.0)).astype(jnp.int32)
    m = jnp.where(t == _BITS_TWO, 0, _K - c_gt)

    idx = lax.broadcasted_iota(jnp.int32, (_ROWS, _COLS), 0) * _COLS + \
        lax.broadcasted_iota(jnp.int32, (_ROWS, _COLS), 1)
    tie = x == t_f
    tie_f = jnp.where(tie, 1.0, 0.0)

    def one_tie():
        return jnp.min(jnp.where(tie, idx, jnp.int32(_N))) + 1

    def multi_tie():
        def idx_step(_, lohi):
            lo2, hi2 = lohi
            mid = lo2 + (hi2 - lo2) // 2
            c = count_ge(jnp.where(idx < mid, tie_f, 0.0)).astype(jnp.int32)
            small = c < m
            return jnp.where(small, mid, lo2), jnp.where(small, hi2, mid)

        _, cut = lax.fori_loop(0, 15, idx_step, (jnp.int32(0), jnp.int32(_N)))
        return cut

    cut = lax.cond(m <= 1, one_tie, multi_tie)

    win = gt | (tie & (idx < cut) & (m > 0))
    y_ref[...] = jnp.where(win, 1.0, 0.0).astype(jnp.float32)


def kernel(x):
    x2 = x.reshape(_ROWS, _COLS)
    y = pl.pallas_call(
        _body,
        out_shape=jax.ShapeDtypeStruct((_ROWS, _COLS), jnp.float32),
    )(x2)
    return y.reshape(_N)


# speedup vs baseline: 5.4702x; 1.4570x over previous
import jax
import jax.numpy as jnp
from jax import lax
from jax.experimental import pallas as pl

_K = 32
_BITS_TWO = 0x40000000
_BITS_INF = 0x7F800000
_N = 32768
_ROWS, _COLS = 256, 128


def _body(x_ref, y_ref):
    x = x_ref[...]
    ones_row = jnp.ones((1, _ROWS), jnp.float32)

    def count_ge(mask_f):
        return jnp.sum(mask_f)

    def val_step(_, lohi):
        lo, hi = lohi
        mid = lo + (hi - lo) // 2
        mid_f = lax.bitcast_convert_type(mid, jnp.float32)
        c = count_ge(jnp.where(x >= mid_f, 1.0, 0.0))
        big = c >= _K
        return jnp.where(big, mid, lo), jnp.where(big, hi, mid)

    lo, _ = lax.fori_loop(0, 30, val_step,
                          (jnp.int32(_BITS_TWO), jnp.int32(_BITS_INF)))
    t = lo
    t_f = lax.bitcast_convert_type(t, jnp.float32)

    gt = x > t_f
    c_gt = count_ge(jnp.where(gt, 1.0, 0.0)).astype(jnp.int32)
    m = jnp.where(t == _BITS_TWO, 0, _K - c_gt)

    idx = lax.broadcasted_iota(jnp.int32, (_ROWS, _COLS), 0) * _COLS + \
        lax.broadcasted_iota(jnp.int32, (_ROWS, _COLS), 1)
    tie = x == t_f
    tie_f = jnp.where(tie, 1.0, 0.0)

    def one_tie():
        return jnp.min(jnp.where(tie, idx, jnp.int32(_N))) + 1

    def multi_tie():
        def idx_step(_, lohi):
            lo2, hi2 = lohi
            mid = lo2 + (hi2 - lo2) // 2
            c = count_ge(jnp.where(idx < mid, tie_f, 0.0)).astype(jnp.int32)
            small = c < m
            return jnp.where(small, mid, lo2), jnp.where(small, hi2, mid)

        _, cut = lax.fori_loop(0, 15, idx_step, (jnp.int32(0), jnp.int32(_N)))
        return cut

    cut = lax.cond(m <= 1, one_tie, multi_tie)

    win = gt | (tie & (idx < cut) & (m > 0))
    y_ref[...] = jnp.where(win, 1.0, 0.0).astype(jnp.float32)


def kernel(x):
    x2 = x.reshape(_ROWS, _COLS)
    y = pl.pallas_call(
        _body,
        out_shape=jax.ShapeDtypeStruct((_ROWS, _COLS), jnp.float32),
    )(x2)
    return y.reshape(_N)


# speedup vs baseline: 7.3569x; 1.3449x over previous
import jax
import jax.numpy as jnp
from jax import lax
from jax.experimental import pallas as pl

_K = 32
_BITS_TWO = 0x40000000
_BITS_INF = 0x7F800000
_N = 32768
_ROWS, _COLS = 256, 128


def _body(x_ref, y_ref):
    x = x_ref[...]
    ones_row = jnp.ones((1, _ROWS), jnp.float32)

    def count_ge(mask_f):
        return jnp.sum(mask_f)

    def val_step(_, lohi):
        lo, hi = lohi
        span = hi - lo
        t1 = lo + span // 4
        t2 = lo + span // 2
        t3 = lo + span // 2 + span // 4
        c1 = count_ge(jnp.where(
            x >= lax.bitcast_convert_type(t1, jnp.float32), 1.0, 0.0))
        c2 = count_ge(jnp.where(
            x >= lax.bitcast_convert_type(t2, jnp.float32), 1.0, 0.0))
        c3 = count_ge(jnp.where(
            x >= lax.bitcast_convert_type(t3, jnp.float32), 1.0, 0.0))
        b3 = c3 >= _K
        b2 = c2 >= _K
        b1 = c1 >= _K
        nlo = jnp.where(b3, t3, jnp.where(b2, t2, jnp.where(b1, t1, lo)))
        nhi = jnp.where(b3, hi, jnp.where(b2, t3, jnp.where(b1, t2, t1)))
        return nlo, nhi

    lo, _ = lax.fori_loop(0, 17, val_step,
                          (jnp.int32(_BITS_TWO), jnp.int32(_BITS_INF)))
    t = lo
    t_f = lax.bitcast_convert_type(t, jnp.float32)

    gt = x > t_f
    c_gt = count_ge(jnp.where(gt, 1.0, 0.0)).astype(jnp.int32)
    m = jnp.where(t == _BITS_TWO, 0, _K - c_gt)

    idx = lax.broadcasted_iota(jnp.int32, (_ROWS, _COLS), 0) * _COLS + \
        lax.broadcasted_iota(jnp.int32, (_ROWS, _COLS), 1)
    tie = x == t_f
    tie_f = jnp.where(tie, 1.0, 0.0)

    def one_tie():
        return jnp.min(jnp.where(tie, idx, jnp.int32(_N))) + 1

    def multi_tie():
        def idx_step(_, lohi):
            lo2, hi2 = lohi
            mid = lo2 + (hi2 - lo2) // 2
            c = count_ge(jnp.where(idx < mid, tie_f, 0.0)).astype(jnp.int32)
            small = c < m
            return jnp.where(small, mid, lo2), jnp.where(small, hi2, mid)

        _, cut = lax.fori_loop(0, 15, idx_step, (jnp.int32(0), jnp.int32(_N)))
        return cut

    cut = lax.cond(m <= 1, one_tie, multi_tie)

    win = gt | (tie & (idx < cut) & (m > 0))
    y_ref[...] = jnp.where(win, 1.0, 0.0).astype(jnp.float32)


def kernel(x):
    x2 = x.reshape(_ROWS, _COLS)
    y = pl.pallas_call(
        _body,
        out_shape=jax.ShapeDtypeStruct((_ROWS, _COLS), jnp.float32),
    )(x2)
    return y.reshape(_N)
